# Initial kernel scaffold; baseline (speedup 1.0000x reference)
import jax
import jax.numpy as jnp
from jax.experimental import pallas as pl


def kernel(h, edge_index, edge_attr, W1e, b1e, W2e, b2e, W1n, b1n, W2n, b2n):
    raise NotImplementedError("write your pallas kernel here")


# speedup vs baseline: 2.0817x; 2.0817x over previous
import functools

import jax
import jax.numpy as jnp
from jax import lax
from jax.experimental import pallas as pl
from jax.experimental.pallas import tpu as pltpu
from jax.experimental.pallas import tpu_sc as plsc

NC, NS = 2, 16
NW = NC * NS
CH = 400


def _sc_gather(h, row, col):
    N, D = h.shape
    E = row.shape[0]
    epw = E // NW
    nchunk = epw // CH
    mesh = plsc.VectorSubcoreMesh(core_axis_name="c", subcore_axis_name="s")

    @functools.partial(
        pl.kernel,
        mesh=mesh,
        out_type=[jax.ShapeDtypeStruct((E, D), jnp.float32),
                  jax.ShapeDtypeStruct((E, D), jnp.float32)],
        scratch_types=[
            pltpu.VMEM((CH,), jnp.int32),
            pltpu.VMEM((CH,), jnp.int32),
            pltpu.VMEM((CH, D), jnp.float32),
            pltpu.VMEM((CH, D), jnp.float32),
            pltpu.SemaphoreType.DMA,
            pltpu.SemaphoreType.DMA,
        ],
    )
    def k(h_hbm, row_hbm, col_hbm, src_hbm, tgt_hbm,
          ri_v, ci_v, sr_v, tg_v, sem1, sem2):
        wid = lax.axis_index("s") * NC + lax.axis_index("c")
        base0 = wid * epw

        @pl.loop(0, nchunk)
        def _(i):
            base = base0 + i * CH
            pltpu.sync_copy(row_hbm.at[pl.ds(base, CH)], ri_v)
            pltpu.sync_copy(col_hbm.at[pl.ds(base, CH)], ci_v)
            a = pltpu.async_copy(h_hbm.at[ri_v], sr_v, sem1)
            b = pltpu.async_copy(h_hbm.at[ci_v], tg_v, sem2)
            a.wait()
            b.wait()
            pltpu.sync_copy(sr_v, src_hbm.at[pl.ds(base, CH)])
            pltpu.sync_copy(tg_v, tgt_hbm.at[pl.ds(base, CH)])

    return k(h, row, col)


def _sc_segment_sum(ef, row, zeros):
    E, D = ef.shape
    Npad = zeros.shape[0]
    SCH = 80
    epc = E // NC
    epw = epc // NS
    nchunk = epw // SCH
    rpt = Npad // NS
    mesh = plsc.VectorSubcoreMesh(core_axis_name="c", subcore_axis_name="s")

    @functools.partial(
        pl.kernel,
        mesh=mesh,
        out_type=jax.ShapeDtypeStruct((NC, Npad, D), jnp.float32),
        scratch_types=[
            pltpu.VMEM((SCH,), jnp.int32),
            pltpu.VMEM((SCH, D), jnp.float32),
            pltpu.VMEM_SHARED((Npad, D), jnp.float32),
        ],
    )
    def k(ef_hbm, row_hbm, zero_hbm, out_hbm, idx_v, ef_v, acc_sh):
        c = lax.axis_index("c")
        s = lax.axis_index("s")
        zbase = s * rpt
        pltpu.sync_copy(zero_hbm.at[pl.ds(zbase, rpt)],
                        acc_sh.at[pl.ds(zbase, rpt)])
        plsc.subcore_barrier()

        base0 = c * epc + s * epw

        @pl.loop(0, nchunk)
        def _(i):
            base = base0 + i * SCH
            pltpu.sync_copy(row_hbm.at[pl.ds(base, SCH)], idx_v)
            pltpu.sync_copy(ef_hbm.at[pl.ds(base, SCH)], ef_v)
            pltpu.sync_copy(ef_v, acc_sh.at[idx_v], add=True)

        plsc.subcore_barrier()
        pltpu.sync_copy(acc_sh.at[pl.ds(zbase, rpt)],
                        out_hbm.at[c, pl.ds(zbase, rpt)])

    return k(ef, row, zeros)


def _tc_edge_mlp(src, tgt, attr, W1e, b1e, W2e, b2e):
    E, D = src.shape
    DE = attr.shape[1]
    H = W2e.shape[0]
    BE = 2000
    hp = jax.lax.Precision.HIGHEST

    def body(src_ref, tgt_ref, attr_ref, w1_ref, b1_ref, w2_ref, b2_ref, out_ref):
        w1 = w1_ref[...]
        z = (jnp.dot(src_ref[...], w1[0:D], precision=hp,
                     preferred_element_type=jnp.float32)
             + jnp.dot(tgt_ref[...], w1[D:2 * D], precision=hp,
                       preferred_element_type=jnp.float32)
             + jnp.dot(attr_ref[...], w1[2 * D:2 * D + DE], precision=hp,
                       preferred_element_type=jnp.float32)
             + b1_ref[...])
        m = z * jax.nn.sigmoid(z)
        y = jnp.dot(m, w2_ref[...], precision=hp,
                    preferred_element_type=jnp.float32) + b2_ref[...]
        out_ref[...] = y * jax.nn.sigmoid(y)

    return pl.pallas_call(
        body,
        grid=(E // BE,),
        in_specs=[
            pl.BlockSpec((BE, D), lambda i: (i, 0)),
            pl.BlockSpec((BE, D), lambda i: (i, 0)),
            pl.BlockSpec((BE, DE), lambda i: (i, 0)),
            pl.BlockSpec((2 * D + DE, H), lambda i: (0, 0)),
            pl.BlockSpec((1, H), lambda i: (0, 0)),
            pl.BlockSpec((H, H), lambda i: (0, 0)),
            pl.BlockSpec((1, H), lambda i: (0, 0)),
        ],
        out_specs=pl.BlockSpec((BE, H), lambda i: (i, 0)),
        out_shape=jax.ShapeDtypeStruct((E, H), jnp.float32),
    )(src, tgt, attr, W1e, b1e.reshape(1, H), W2e, b2e.reshape(1, H))


def _tc_node_mlp(h, p0, p1, W1n, b1n, W2n, b2n):
    N, D = h.shape
    H = p0.shape[1]
    DO = W2n.shape[1]
    BN = 2000
    hp = jax.lax.Precision.HIGHEST

    def body(h_ref, p0_ref, p1_ref, w1_ref, b1_ref, w2_ref, b2_ref, out_ref):
        agg = p0_ref[...] + p1_ref[...]
        w1 = w1_ref[...]
        z = (jnp.dot(h_ref[...], w1[0:D], precision=hp,
                     preferred_element_type=jnp.float32)
             + jnp.dot(agg, w1[D:D + H], precision=hp,
                       preferred_element_type=jnp.float32)
             + b1_ref[...])
        t = z * jax.nn.sigmoid(z)
        out_ref[...] = jnp.dot(t, w2_ref[...], precision=hp,
                               preferred_element_type=jnp.float32) + b2_ref[...]

    return pl.pallas_call(
        body,
        grid=(N // BN,),
        in_specs=[
            pl.BlockSpec((BN, D), lambda i: (i, 0)),
            pl.BlockSpec((BN, H), lambda i: (i, 0)),
            pl.BlockSpec((BN, H), lambda i: (i, 0)),
            pl.BlockSpec((D + H, H), lambda i: (0, 0)),
            pl.BlockSpec((1, H), lambda i: (0, 0)),
            pl.BlockSpec((H, DO), lambda i: (0, 0)),
            pl.BlockSpec((1, DO), lambda i: (0, 0)),
        ],
        out_specs=pl.BlockSpec((BN, DO), lambda i: (i, 0)),
        out_shape=jax.ShapeDtypeStruct((N, DO), jnp.float32),
    )(h, p0, p1, W1n, b1n.reshape(1, H), W2n, b2n.reshape(1, DO))


def kernel(h, edge_index, edge_attr, W1e, b1e, W2e, b2e, W1n, b1n, W2n, b2n):
    N = h.shape[0]
    row = edge_index[0]
    col = edge_index[1]
    src, tgt = _sc_gather(h, row, col)
    ef = _tc_edge_mlp(src, tgt, edge_attr, W1e, b1e, W2e, b2e)
    Npad = ((N + 8 * NS - 1) // (8 * NS)) * (8 * NS)
    zeros = jnp.zeros((Npad, h.shape[1]), jnp.float32)
    p = _sc_segment_sum(ef, row, zeros)
    return _tc_node_mlp(h, p[0, :N], p[1, :N], W1n, b1n, W2n, b2n)


# speedup vs baseline: 3.4112x; 1.6386x over previous
import functools

import jax
import jax.numpy as jnp
from jax import lax
from jax.experimental import pallas as pl
from jax.experimental.pallas import tpu as pltpu
from jax.experimental.pallas import tpu_sc as plsc

NC, NS = 2, 16
NW = NC * NS
CH = 400


def _tc_preproject(h, W1e):
    N, D = h.shape
    H = W1e.shape[1]

    def body(h_ref, w1_ref, hs_ref, ht_ref):
        w1 = w1_ref[...]
        hv = h_ref[...]
        hs_ref[...] = jnp.dot(hv, w1[0:D], preferred_element_type=jnp.float32)
        ht_ref[...] = jnp.dot(hv, w1[D:2 * D], preferred_element_type=jnp.float32)

    return pl.pallas_call(
        body,
        grid=(1,),
        in_specs=[
            pl.BlockSpec((N, D), lambda i: (0, 0)),
            pl.BlockSpec(W1e.shape, lambda i: (0, 0)),
        ],
        out_specs=[
            pl.BlockSpec((N, H), lambda i: (0, 0)),
            pl.BlockSpec((N, H), lambda i: (0, 0)),
        ],
        out_shape=[jax.ShapeDtypeStruct((N, H), jnp.float32),
                   jax.ShapeDtypeStruct((N, H), jnp.float32)],
    )(h, W1e)


def _sc_gather(hs, ht, row, col):
    N, D = hs.shape
    E = row.shape[0]
    epw = E // NW
    nchunk = epw // CH
    mesh = plsc.VectorSubcoreMesh(core_axis_name="c", subcore_axis_name="s")

    @functools.partial(
        pl.kernel,
        mesh=mesh,
        out_type=[jax.ShapeDtypeStruct((E, D), jnp.float32),
                  jax.ShapeDtypeStruct((E, D), jnp.float32)],
        scratch_types=[
            pltpu.VMEM((CH,), jnp.int32),
            pltpu.VMEM((CH,), jnp.int32),
            pltpu.VMEM((CH, D), jnp.float32),
            pltpu.VMEM((CH, D), jnp.float32),
            pltpu.SemaphoreType.DMA,
            pltpu.SemaphoreType.DMA,
        ],
    )
    def k(hs_hbm, ht_hbm, row_hbm, col_hbm, src_hbm, tgt_hbm,
          ri_v, ci_v, sr_v, tg_v, sem1, sem2):
        wid = lax.axis_index("s") * NC + lax.axis_index("c")
        base0 = wid * epw

        @pl.loop(0, nchunk)
        def _(i):
            base = base0 + i * CH
            pltpu.sync_copy(row_hbm.at[pl.ds(base, CH)], ri_v)
            pltpu.sync_copy(col_hbm.at[pl.ds(base, CH)], ci_v)
            a = pltpu.async_copy(hs_hbm.at[ri_v], sr_v, sem1)
            b = pltpu.async_copy(ht_hbm.at[ci_v], tg_v, sem2)
            a.wait()
            b.wait()
            pltpu.sync_copy(sr_v, src_hbm.at[pl.ds(base, CH)])
            pltpu.sync_copy(tg_v, tgt_hbm.at[pl.ds(base, CH)])

    return k(hs, ht, row, col)


def _sc_segment_sum(ef, row, zeros):
    E, D = ef.shape
    Npad = zeros.shape[0]
    SCH = 80
    epc = E // NC
    epw = epc // NS
    nchunk = epw // SCH
    rpt = Npad // NS
    mesh = plsc.VectorSubcoreMesh(core_axis_name="c", subcore_axis_name="s")

    @functools.partial(
        pl.kernel,
        mesh=mesh,
        out_type=jax.ShapeDtypeStruct((NC, Npad, D), jnp.float32),
        scratch_types=[
            pltpu.VMEM((SCH,), jnp.int32),
            pltpu.VMEM((SCH, D), jnp.float32),
            pltpu.VMEM_SHARED((Npad, D), jnp.float32),
        ],
    )
    def k(ef_hbm, row_hbm, zero_hbm, out_hbm, idx_v, ef_v, acc_sh):
        c = lax.axis_index("c")
        s = lax.axis_index("s")
        zbase = s * rpt
        pltpu.sync_copy(zero_hbm.at[pl.ds(zbase, rpt)],
                        acc_sh.at[pl.ds(zbase, rpt)])
        plsc.subcore_barrier()

        base0 = c * epc + s * epw

        @pl.loop(0, nchunk)
        def _(i):
            base = base0 + i * SCH
            pltpu.sync_copy(row_hbm.at[pl.ds(base, SCH)], idx_v)
            pltpu.sync_copy(ef_hbm.at[pl.ds(base, SCH)], ef_v)
            pltpu.sync_copy(ef_v, acc_sh.at[idx_v], add=True)

        plsc.subcore_barrier()
        pltpu.sync_copy(acc_sh.at[pl.ds(zbase, rpt)],
                        out_hbm.at[c, pl.ds(zbase, rpt)])

    return k(ef, row, zeros)


def _tc_edge_mlp(src, tgt, attr, W1e, b1e, W2e, b2e):
    E, D = src.shape
    DE = attr.shape[1]
    H = W2e.shape[0]
    BE = 2000

    def body(src_ref, tgt_ref, attr_ref, w1c_ref, b1_ref, w2_ref, b2_ref, out_ref):
        z = (src_ref[...]
             + tgt_ref[...]
             + jnp.dot(attr_ref[...], w1c_ref[...],
                       preferred_element_type=jnp.float32)
             + b1_ref[...])
        m = z * jax.nn.sigmoid(z)
        y = jnp.dot(m, w2_ref[...],
                    preferred_element_type=jnp.float32) + b2_ref[...]
        out_ref[...] = y * jax.nn.sigmoid(y)

    return pl.pallas_call(
        body,
        grid=(E // BE,),
        in_specs=[
            pl.BlockSpec((BE, D), lambda i: (i, 0)),
            pl.BlockSpec((BE, D), lambda i: (i, 0)),
            pl.BlockSpec((BE, DE), lambda i: (i, 0)),
            pl.BlockSpec((DE, H), lambda i: (0, 0)),
            pl.BlockSpec((1, H), lambda i: (0, 0)),
            pl.BlockSpec((H, H), lambda i: (0, 0)),
            pl.BlockSpec((1, H), lambda i: (0, 0)),
        ],
        out_specs=pl.BlockSpec((BE, H), lambda i: (i, 0)),
        out_shape=jax.ShapeDtypeStruct((E, H), jnp.float32),
    )(src, tgt, attr, W1e[2 * D:2 * D + DE], b1e.reshape(1, H), W2e,
      b2e.reshape(1, H))


def _tc_node_mlp(h, p0, p1, W1n, b1n, W2n, b2n):
    N, D = h.shape
    H = p0.shape[1]
    DO = W2n.shape[1]
    BN = 2000

    def body(h_ref, p0_ref, p1_ref, w1_ref, b1_ref, w2_ref, b2_ref, out_ref):
        agg = p0_ref[...] + p1_ref[...]
        w1 = w1_ref[...]
        z = (jnp.dot(h_ref[...], w1[0:D], preferred_element_type=jnp.float32)
             + jnp.dot(agg, w1[D:D + H], preferred_element_type=jnp.float32)
             + b1_ref[...])
        t = z * jax.nn.sigmoid(z)
        out_ref[...] = jnp.dot(t, w2_ref[...],
                               preferred_element_type=jnp.float32) + b2_ref[...]

    return pl.pallas_call(
        body,
        grid=(N // BN,),
        in_specs=[
            pl.BlockSpec((BN, D), lambda i: (i, 0)),
            pl.BlockSpec((BN, H), lambda i: (i, 0)),
            pl.BlockSpec((BN, H), lambda i: (i, 0)),
            pl.BlockSpec((D + H, H), lambda i: (0, 0)),
            pl.BlockSpec((1, H), lambda i: (0, 0)),
            pl.BlockSpec((H, DO), lambda i: (0, 0)),
            pl.BlockSpec((1, DO), lambda i: (0, 0)),
        ],
        out_specs=pl.BlockSpec((BN, DO), lambda i: (i, 0)),
        out_shape=jax.ShapeDtypeStruct((N, DO), jnp.float32),
    )(h, p0, p1, W1n, b1n.reshape(1, H), W2n, b2n.reshape(1, DO))


def kernel(h, edge_index, edge_attr, W1e, b1e, W2e, b2e, W1n, b1n, W2n, b2n):
    N = h.shape[0]
    row = edge_index[0]
    col = edge_index[1]
    hs, ht = _tc_preproject(h, W1e)
    src, tgt = _sc_gather(hs, ht, row, col)
    ef = _tc_edge_mlp(src, tgt, edge_attr, W1e, b1e, W2e, b2e)
    Npad = ((N + 8 * NS - 1) // (8 * NS)) * (8 * NS)
    zeros = jnp.zeros((Npad, h.shape[1]), jnp.float32)
    p = _sc_segment_sum(ef, row, zeros)
    return _tc_node_mlp(h, p[0, :N], p[1, :N], W1n, b1n, W2n, b2n)


# speedup vs baseline: 3.9074x; 1.1455x over previous
import functools

import jax
import jax.numpy as jnp
from jax import lax
from jax.experimental import pallas as pl
from jax.experimental.pallas import tpu as pltpu
from jax.experimental.pallas import tpu_sc as plsc

NC, NS = 2, 16
NW = NC * NS
NSPLIT = 2


def _tc_preproject(h, W1e):
    N, D = h.shape
    H = W1e.shape[1]

    def body(h_ref, w1_ref, hs_ref, ht_ref):
        w1 = w1_ref[...]
        hv = h_ref[...]
        hs_ref[...] = jnp.dot(hv, w1[0:D], preferred_element_type=jnp.float32)
        ht_ref[...] = jnp.dot(hv, w1[D:2 * D], preferred_element_type=jnp.float32)

    return pl.pallas_call(
        body,
        grid=(1,),
        in_specs=[
            pl.BlockSpec((N, D), lambda i: (0, 0)),
            pl.BlockSpec(W1e.shape, lambda i: (0, 0)),
        ],
        out_specs=[
            pl.BlockSpec((N, H), lambda i: (0, 0)),
            pl.BlockSpec((N, H), lambda i: (0, 0)),
        ],
        out_shape=[jax.ShapeDtypeStruct((N, H), jnp.float32),
                   jax.ShapeDtypeStruct((N, H), jnp.float32)],
    )(h, W1e)


def _sc_gather(hs, ht, row, col):
    N, D = hs.shape
    E = row.shape[0]
    epw = E // NW
    CH = 200
    nchunk = epw // CH
    mesh = plsc.VectorSubcoreMesh(core_axis_name="c", subcore_axis_name="s")

    @functools.partial(
        pl.kernel,
        mesh=mesh,
        out_type=[jax.ShapeDtypeStruct((E, D), jnp.float32),
                  jax.ShapeDtypeStruct((E, D), jnp.float32)],
        scratch_types=[
            pltpu.VMEM((CH,), jnp.int32),
            pltpu.VMEM((CH,), jnp.int32),
            pltpu.VMEM((CH, D), jnp.float32),
            pltpu.VMEM((CH, D), jnp.float32),
            pltpu.SemaphoreType.DMA,
            pltpu.SemaphoreType.DMA,
        ],
    )
    def k(hs_hbm, ht_hbm, row_hbm, col_hbm, src_hbm, tgt_hbm,
          ri_v, ci_v, sr_v, tg_v, sem1, sem2):
        wid = lax.axis_index("s") * NC + lax.axis_index("c")
        base0 = wid * epw

        @pl.loop(0, nchunk)
        def _(i):
            base = base0 + i * CH
            pltpu.sync_copy(row_hbm.at[pl.ds(base, CH)], ri_v)
            pltpu.sync_copy(col_hbm.at[pl.ds(base, CH)], ci_v)
            a = pltpu.async_copy(hs_hbm.at[ri_v], sr_v, sem1)
            b = pltpu.async_copy(ht_hbm.at[ci_v], tg_v, sem2)
            a.wait()
            b.wait()
            pltpu.sync_copy(sr_v, src_hbm.at[pl.ds(base, CH)])
            pltpu.sync_copy(tg_v, tgt_hbm.at[pl.ds(base, CH)])

    return k(hs, ht, row, col)


def _sc_segment_sum(ef_halves, row, zeros):
    Eh, D = ef_halves[0].shape
    E = Eh * NC
    Npad = zeros.shape[0]
    SCH = 80
    epw = Eh // NS
    nchunk = epw // SCH
    rpt = Npad // NS
    mesh = plsc.VectorSubcoreMesh(core_axis_name="c", subcore_axis_name="s")

    @functools.partial(
        pl.kernel,
        mesh=mesh,
        out_type=jax.ShapeDtypeStruct((NC, Npad, D), jnp.float32),
        scratch_types=[
            pltpu.VMEM((SCH,), jnp.int32),
            pltpu.VMEM((SCH, D), jnp.float32),
            pltpu.VMEM_SHARED((Npad, D), jnp.float32),
        ],
    )
    def k(ef0_hbm, ef1_hbm, row_hbm, zero_hbm, out_hbm, idx_v, ef_v, acc_sh):
        c = lax.axis_index("c")
        s = lax.axis_index("s")
        zbase = s * rpt
        pltpu.sync_copy(zero_hbm.at[pl.ds(zbase, rpt)],
                        acc_sh.at[pl.ds(zbase, rpt)])
        plsc.subcore_barrier()

        for cc, ef_hbm in enumerate((ef0_hbm, ef1_hbm)):
            @pl.when(c == cc)
            def _():
                base0 = s * epw

                @pl.loop(0, nchunk)
                def _(i):
                    base = base0 + i * SCH
                    pltpu.sync_copy(row_hbm.at[pl.ds(cc * Eh + base, SCH)],
                                    idx_v)
                    pltpu.sync_copy(ef_hbm.at[pl.ds(base, SCH)], ef_v)
                    pltpu.sync_copy(ef_v, acc_sh.at[idx_v], add=True)

        plsc.subcore_barrier()
        pltpu.sync_copy(acc_sh.at[pl.ds(zbase, rpt)],
                        out_hbm.at[c, pl.ds(zbase, rpt)])

    return k(ef_halves[0], ef_halves[1], row, zeros)


def _tc_edge_mlp(src, tgt, attr_t, W1e, b1e, W2e, b2e):
    E, D = src.shape
    DE = attr_t.shape[0]
    H = W2e.shape[0]
    BE = 6400

    def body(src_ref, tgt_ref, attr_ref, w1c_ref, b1_ref, w2_ref, b2_ref, out_ref):
        za = lax.dot_general(attr_ref[...], w1c_ref[...],
                             (((0,), (0,)), ((), ())),
                             preferred_element_type=jnp.float32)
        z = src_ref[...] + tgt_ref[...] + za + b1_ref[...]
        m = z * jax.nn.sigmoid(z)
        y = jnp.dot(m, w2_ref[...],
                    preferred_element_type=jnp.float32) + b2_ref[...]
        out_ref[...] = y * jax.nn.sigmoid(y)

    return pl.pallas_call(
        body,
        grid=(E // BE,),
        in_specs=[
            pl.BlockSpec((BE, D), lambda i: (i, 0)),
            pl.BlockSpec((BE, D), lambda i: (i, 0)),
            pl.BlockSpec((DE, BE), lambda i: (0, i)),
            pl.BlockSpec((DE, H), lambda i: (0, 0)),
            pl.BlockSpec((1, H), lambda i: (0, 0)),
            pl.BlockSpec((H, H), lambda i: (0, 0)),
            pl.BlockSpec((1, H), lambda i: (0, 0)),
        ],
        out_specs=pl.BlockSpec((BE, H), lambda i: (i, 0)),
        out_shape=jax.ShapeDtypeStruct((E, H), jnp.float32),
    )(src, tgt, attr_t, W1e[2 * D:2 * D + DE], b1e.reshape(1, H), W2e,
      b2e.reshape(1, H))


def _tc_node_mlp(h, p0, p1, W1n, b1n, W2n, b2n):
    N, D = h.shape
    H = p0.shape[1]
    DO = W2n.shape[1]
    BN = 2000

    def body(h_ref, p0_ref, p1_ref, w1_ref, b1_ref, w2_ref, b2_ref, out_ref):
        agg = p0_ref[...] + p1_ref[...]
        w1 = w1_ref[...]
        z = (jnp.dot(h_ref[...], w1[0:D], preferred_element_type=jnp.float32)
             + jnp.dot(agg, w1[D:D + H], preferred_element_type=jnp.float32)
             + b1_ref[...])
        t = z * jax.nn.sigmoid(z)
        out_ref[...] = jnp.dot(t, w2_ref[...],
                               preferred_element_type=jnp.float32) + b2_ref[...]

    return pl.pallas_call(
        body,
        grid=(N // BN,),
        in_specs=[
            pl.BlockSpec((BN, D), lambda i: (i, 0)),
            pl.BlockSpec((BN, H), lambda i: (i, 0)),
            pl.BlockSpec((BN, H), lambda i: (i, 0)),
            pl.BlockSpec((D + H, H), lambda i: (0, 0)),
            pl.BlockSpec((1, H), lambda i: (0, 0)),
            pl.BlockSpec((H, DO), lambda i: (0, 0)),
            pl.BlockSpec((1, DO), lambda i: (0, 0)),
        ],
        out_specs=pl.BlockSpec((BN, DO), lambda i: (i, 0)),
        out_shape=jax.ShapeDtypeStruct((N, DO), jnp.float32),
    )(h, p0, p1, W1n, b1n.reshape(1, H), W2n, b2n.reshape(1, DO))


def kernel(h, edge_index, edge_attr, W1e, b1e, W2e, b2e, W1n, b1n, W2n, b2n):
    N = h.shape[0]
    E = edge_index.shape[1]
    Eh = E // NSPLIT
    row = edge_index[0]
    col = edge_index[1]
    attr_t = edge_attr.T
    hs, ht = _tc_preproject(h, W1e)
    ef_halves = []
    for ci in range(NSPLIT):
        lo = ci * Eh
        src, tgt = _sc_gather(hs, ht, row[lo:lo + Eh], col[lo:lo + Eh])
        ef_halves.append(_tc_edge_mlp(src, tgt, attr_t[:, lo:lo + Eh],
                                      W1e, b1e, W2e, b2e))
    Npad = ((N + 8 * NS - 1) // (8 * NS)) * (8 * NS)
    zeros = jnp.zeros((Npad, h.shape[1]), jnp.float32)
    p = _sc_segment_sum(ef_halves, row, zeros)
    return _tc_node_mlp(h, p[0, :N], p[1, :N], W1n, b1n, W2n, b2n)
